# baseline (speedup 1.0000x reference)
import functools

import jax
import jax.numpy as jnp
from jax import lax
from jax.experimental import pallas as pl
from jax.experimental.pallas import tpu as pltpu
from jax.experimental.pallas import tpu_sc as plsc

N = 10000
D = 128
E = 320000
EPS = 1e-5

NC = 2
NS = 16
L = 16
NW = NC * NS
EPW = E // NW
CH = 80
NCH = EPW // CH
DSTH = 64
NCHP = 2 * DSTH
RPT = 624
TAIL0 = RPT * NS
TAIL = N - TAIL0

_mesh = plsc.VectorSubcoreMesh(
    core_axis_name="c", subcore_axis_name="s", num_cores=NC, num_subcores=NS)


@functools.partial(
    pl.kernel,
    out_type=jax.ShapeDtypeStruct((NC, N, D), jnp.float32),
    mesh=_mesh,
    scratch_types=[
        pltpu.VMEM((EPW,), jnp.int32),
        pltpu.VMEM((DSTH, CH), jnp.int32),
        pltpu.VMEM((EPW,), jnp.float32),
        pltpu.VMEM((CH, D), jnp.float32),
        pltpu.VMEM((CH, D), jnp.float32),
        pltpu.VMEM_SHARED((N, D), jnp.float32),
        pltpu.SemaphoreType.DMA,
        pltpu.SemaphoreType.DMA,
        pltpu.SemaphoreType.DMA,
        pltpu.SemaphoreType.DMA,
        pltpu.SemaphoreType.DMA,
    ],
)
def _sc_agg(h_hbm, src_hbm, dst_hbm, ew_hbm, zeros_hbm, out_hbm,
            src_v, dst_v, ew_v, rows0, rows1, acc, sem_s,
            sem_g0, sem_g0b, sem_g1, sem_g1b):
    cid = lax.axis_index("c")
    sid = lax.axis_index("s")
    wid = sid * NC + cid
    H = CH // 2

    class _gather_desc:
        def __init__(self, c, rows, sem, semb):
            off = pl.multiple_of(c * CH, 8)
            self.a = pltpu.make_async_copy(
                h_hbm.at[src_v.at[pl.ds(off, H)]],
                rows.at[pl.ds(0, H)], sem)
            self.b = pltpu.make_async_copy(
                h_hbm.at[src_v.at[pl.ds(pl.multiple_of(off + H, 8), H)]],
                rows.at[pl.ds(H, H)], semb)

        def start(self):
            self.a.start()
            self.b.start()

        def wait(self):
            self.a.wait()
            self.b.wait()

    def _scale(rows, off):
        for r0 in range(0, CH, L):
            wv = ew_v[pl.ds(pl.multiple_of(off + r0, 8), L)]
            for i in range(L):
                w = jnp.full((L,), wv[i], jnp.float32)
                for c in range(D // L):
                    rows[r0 + i, pl.ds(c * L, L)] = (
                        rows[r0 + i, pl.ds(c * L, L)] * w)

    def _scatter(rows, c):
        pltpu.sync_copy(rows, acc.at[dst_v.at[lax.rem(c, DSTH)]], add=True)

    a1 = pltpu.async_copy(src_hbm.at[pl.ds(wid * EPW, EPW)], src_v, sem_s)
    a2 = pltpu.async_copy(ew_hbm.at[pl.ds(wid * EPW, EPW)], ew_v, sem_s)
    a3 = pltpu.async_copy(dst_hbm.at[wid, pl.ds(0, DSTH)], dst_v, sem_s)

    pltpu.sync_copy(zeros_hbm.at[pl.ds(sid * RPT, RPT)],
                    acc.at[pl.ds(sid * RPT, RPT)])

    @pl.when(sid == 0)
    def _():
        pltpu.sync_copy(zeros_hbm.at[pl.ds(TAIL0, TAIL)],
                        acc.at[pl.ds(TAIL0, TAIL)])

    a1.wait()
    a2.wait()
    a3.wait()
    _gather_desc(0, rows0, sem_g0, sem_g0b).start()
    plsc.subcore_barrier()

    def body(i, carry):
        c0 = 2 * i
        c1 = c0 + 1

        @pl.when(i == DSTH // 2)
        def _():
            pltpu.sync_copy(dst_hbm.at[wid, pl.ds(DSTH, DSTH)], dst_v)

        _gather_desc(c0, rows0, sem_g0, sem_g0b).wait()
        _gather_desc(c1, rows1, sem_g1, sem_g1b).start()
        _scale(rows0, c0 * CH)
        _scatter(rows0, c0)

        _gather_desc(c1, rows1, sem_g1, sem_g1b).wait()
        _gather_desc(c0 + 2, rows0, sem_g0, sem_g0b).start()
        _scale(rows1, c1 * CH)
        _scatter(rows1, c1)
        return carry

    lax.fori_loop(0, NCH // 2, body, 0)

    _gather_desc(NCH - 1, rows0, sem_g0, sem_g0b).wait()
    _scale(rows0, (NCH - 1) * CH)
    _scatter(rows0, NCH - 1)
    plsc.subcore_barrier()
    pltpu.sync_copy(acc.at[pl.ds(sid * RPT, RPT)],
                    out_hbm.at[cid, pl.ds(sid * RPT, RPT)])

    @pl.when(sid == 0)
    def _():
        pltpu.sync_copy(acc.at[pl.ds(TAIL0, TAIL)],
                        out_hbm.at[cid, pl.ds(TAIL0, TAIL)])


def _tc_in_body(x_ref, w_ref, b_ref, o_ref):
    o_ref[...] = jnp.maximum(
        jnp.dot(x_ref[...], w_ref[...], preferred_element_type=jnp.float32)
        + b_ref[...], 0.0)


def _tc_root_body(h_ref, wroot_ref, b_ref, o_ref):
    o_ref[...] = jnp.dot(h_ref[...], wroot_ref[...],
                         preferred_element_type=jnp.float32) + b_ref[...]


def _tc_mix_body(p_ref, root_ref, wrel_ref, g_ref, be_ref, o_ref, *, relu):
    agg = p_ref[0] + p_ref[1]
    t = (jnp.dot(agg, wrel_ref[...], preferred_element_type=jnp.float32)
         + root_ref[...])
    mean = jnp.mean(t, axis=0, keepdims=True)
    var = jnp.mean(jnp.square(t - mean), axis=0, keepdims=True)
    t = (t - mean) / jnp.sqrt(var + EPS) * g_ref[...] + be_ref[...]
    if relu:
        t = jnp.maximum(t, 0.0)
    o_ref[...] = t


_tc_in = pl.pallas_call(
    _tc_in_body, out_shape=jax.ShapeDtypeStruct((N, D), jnp.float32))

_tc_root = pl.pallas_call(
    _tc_root_body, out_shape=jax.ShapeDtypeStruct((N, D), jnp.float32))


def _tc_mix(p, root, wrel, gamma, beta, relu):
    body = functools.partial(_tc_mix_body, relu=relu)
    return pl.pallas_call(
        body, out_shape=jax.ShapeDtypeStruct((N, D), jnp.float32))(
            p, root, wrel, gamma.reshape(1, D), beta.reshape(1, D))


def kernel(x, adj, features, W_in, b_in, W_rel1, b_rel1, W_root1,
           W_rel2, b_rel2, W_root2, gamma1, beta1):
    src = adj[0]
    dst = jnp.pad(adj[1].reshape(NW, NCH, CH),
                  ((0, 0), (0, NCHP - NCH), (0, 0)))
    ew = features
    zeros = jnp.zeros((N, D), jnp.float32)

    h0 = _tc_in(x, W_in, b_in.reshape(1, D))
    p1 = _sc_agg(h0, src, dst, ew, zeros)
    root1 = _tc_root(h0, W_root1, b_rel1.reshape(1, D))
    h1 = _tc_mix(p1, root1, W_rel1, gamma1, beta1, relu=True)
    p2 = _sc_agg(h1, src, dst, ew, zeros)
    root2 = _tc_root(h1, W_root2, b_rel2.reshape(1, D))
    out = _tc_mix(p2, root2, W_rel2, gamma1, beta1, relu=False)
    return out


# speedup vs baseline: 1.0061x; 1.0061x over previous
import functools

import jax
import jax.numpy as jnp
from jax import lax
from jax.experimental import pallas as pl
from jax.experimental.pallas import tpu as pltpu
from jax.experimental.pallas import tpu_sc as plsc

N = 10000
D = 128
E = 320000
EPS = 1e-5

NC = 2
NS = 16
L = 16
NW = NC * NS
EPW = E // NW
CH = 80
NCH = EPW // CH
DSTH = 64
NCHP = 2 * DSTH
RPT = 624
TAIL0 = RPT * NS
TAIL = N - TAIL0

_mesh = plsc.VectorSubcoreMesh(
    core_axis_name="c", subcore_axis_name="s", num_cores=NC, num_subcores=NS)


@functools.partial(
    pl.kernel,
    out_type=jax.ShapeDtypeStruct((NC, N, D), jnp.float32),
    mesh=_mesh,
    scratch_types=[
        pltpu.VMEM((EPW,), jnp.int32),
        pltpu.VMEM((DSTH, CH), jnp.int32),
        pltpu.VMEM((EPW,), jnp.float32),
        pltpu.VMEM((CH, D), jnp.float32),
        pltpu.VMEM((CH, D), jnp.float32),
        pltpu.VMEM_SHARED((N, D), jnp.float32),
        pltpu.SemaphoreType.DMA,
        pltpu.SemaphoreType.DMA,
        pltpu.SemaphoreType.DMA,
        pltpu.SemaphoreType.DMA,
        pltpu.SemaphoreType.DMA,
    ],
)
def _sc_agg(h_hbm, src_hbm, dst_hbm, ew_hbm, zeros_hbm, out_hbm,
            src_v, dst_v, ew_v, rows0, rows1, acc, sem_s,
            sem_g0, sem_g0b, sem_g1, sem_g1b):
    cid = lax.axis_index("c")
    sid = lax.axis_index("s")
    wid = sid * NC + cid
    H = CH // 2

    class _gather_desc:
        def __init__(self, c, rows, sem, semb):
            off = pl.multiple_of(c * CH, 8)
            self.a = pltpu.make_async_copy(
                h_hbm.at[src_v.at[pl.ds(off, H)]],
                rows.at[pl.ds(0, H)], sem)
            self.b = pltpu.make_async_copy(
                h_hbm.at[src_v.at[pl.ds(pl.multiple_of(off + H, 8), H)]],
                rows.at[pl.ds(H, H)], semb)

        def start(self):
            self.a.start()
            self.b.start()

        def wait(self):
            self.a.wait()
            self.b.wait()

    def _scale(rows, off):
        for r0 in range(0, CH, L):
            wv = ew_v[pl.ds(pl.multiple_of(off + r0, 8), L)]
            for i in range(L):
                w = jnp.full((L,), wv[i], jnp.float32)
                for c in range(D // L):
                    rows[r0 + i, pl.ds(c * L, L)] = (
                        rows[r0 + i, pl.ds(c * L, L)] * w)

    def _scatter(rows, c):
        pltpu.sync_copy(rows, acc.at[dst_v.at[lax.rem(c, DSTH)]], add=True)

    a1 = pltpu.async_copy(src_hbm.at[pl.ds(wid * EPW, EPW)], src_v, sem_s)
    a2 = pltpu.async_copy(ew_hbm.at[pl.ds(wid * EPW, EPW)], ew_v, sem_s)
    a3 = pltpu.async_copy(dst_hbm.at[wid, pl.ds(0, DSTH)], dst_v, sem_s)

    pltpu.sync_copy(zeros_hbm.at[pl.ds(sid * RPT, RPT)],
                    acc.at[pl.ds(sid * RPT, RPT)])

    @pl.when(sid == 0)
    def _():
        pltpu.sync_copy(zeros_hbm.at[pl.ds(TAIL0, TAIL)],
                        acc.at[pl.ds(TAIL0, TAIL)])

    a1.wait()
    a2.wait()
    a3.wait()
    _gather_desc(0, rows0, sem_g0, sem_g0b).start()
    plsc.subcore_barrier()

    def body(i, carry):
        c0 = 2 * i
        c1 = c0 + 1

        @pl.when(i == DSTH // 2)
        def _():
            pltpu.sync_copy(dst_hbm.at[wid, pl.ds(DSTH, DSTH)], dst_v)

        _gather_desc(c0, rows0, sem_g0, sem_g0b).wait()
        _gather_desc(c1, rows1, sem_g1, sem_g1b).start()
        _scale(rows0, c0 * CH)
        _scatter(rows0, c0)

        _gather_desc(c1, rows1, sem_g1, sem_g1b).wait()
        _gather_desc(c0 + 2, rows0, sem_g0, sem_g0b).start()
        _scale(rows1, c1 * CH)
        _scatter(rows1, c1)
        return carry

    lax.fori_loop(0, NCH // 2, body, 0)

    _gather_desc(NCH - 1, rows0, sem_g0, sem_g0b).wait()
    _scale(rows0, (NCH - 1) * CH)
    _scatter(rows0, NCH - 1)
    plsc.subcore_barrier()
    pltpu.sync_copy(acc.at[pl.ds(sid * RPT, RPT)],
                    out_hbm.at[cid, pl.ds(sid * RPT, RPT)])

    @pl.when(sid == 0)
    def _():
        pltpu.sync_copy(acc.at[pl.ds(TAIL0, TAIL)],
                        out_hbm.at[cid, pl.ds(TAIL0, TAIL)])


def _tc_in_body(x_ref, w_ref, b_ref, o_ref):
    o_ref[...] = jnp.maximum(
        jnp.dot(x_ref[...], w_ref[...], preferred_element_type=jnp.float32)
        + b_ref[...], 0.0)


def _tc_mix_body(p_ref, h_ref, wrel_ref, brel_ref, wroot_ref, g_ref, be_ref,
                 o_ref, *, relu):
    agg = p_ref[0] + p_ref[1]
    t = (jnp.dot(agg, wrel_ref[...], preferred_element_type=jnp.float32)
         + brel_ref[...]
         + jnp.dot(h_ref[...], wroot_ref[...], preferred_element_type=jnp.float32))
    mean = jnp.mean(t, axis=0, keepdims=True)
    var = jnp.mean(jnp.square(t - mean), axis=0, keepdims=True)
    t = (t - mean) / jnp.sqrt(var + EPS) * g_ref[...] + be_ref[...]
    if relu:
        t = jnp.maximum(t, 0.0)
    o_ref[...] = t


_tc_in = pl.pallas_call(
    _tc_in_body, out_shape=jax.ShapeDtypeStruct((N, D), jnp.float32))


def _tc_mix(p, h, wrel, brel, wroot, gamma, beta, relu):
    body = functools.partial(_tc_mix_body, relu=relu)
    return pl.pallas_call(
        body, out_shape=jax.ShapeDtypeStruct((N, D), jnp.float32))(
            p, h, wrel, brel.reshape(1, D), wroot,
            gamma.reshape(1, D), beta.reshape(1, D))


def kernel(x, adj, features, W_in, b_in, W_rel1, b_rel1, W_root1,
           W_rel2, b_rel2, W_root2, gamma1, beta1):
    src = adj[0]
    dst = jnp.pad(adj[1].reshape(NW, NCH, CH),
                  ((0, 0), (0, NCHP - NCH), (0, 0)))
    ew = features
    zeros = jnp.zeros((N, D), jnp.float32)

    h0 = _tc_in(x, W_in, b_in.reshape(1, D))
    p1 = _sc_agg(h0, src, dst, ew, zeros)
    h1 = _tc_mix(p1, h0, W_rel1, b_rel1, W_root1, gamma1, beta1, relu=True)
    p2 = _sc_agg(h1, src, dst, ew, zeros)
    out = _tc_mix(p2, h1, W_rel2, b_rel2, W_root2, gamma1, beta1, relu=False)
    return out
